# Initial kernel scaffold; baseline (speedup 1.0000x reference)
import jax
import jax.numpy as jnp
from jax.experimental import pallas as pl


def kernel(features, edge_index, W1, b1, W2, b2, W3, b3, Wo, bo):
    raise NotImplementedError("write your pallas kernel here")


# speedup vs baseline: 74.8688x; 74.8688x over previous
import functools

import jax
import jax.numpy as jnp
from jax import lax
from jax.experimental import pallas as pl
from jax.experimental.pallas import tpu as pltpu
from jax.experimental.pallas import tpu_sc as plsc

NN = 10000
EE = 320000
DD = 128
HH = 128
CC = 64
NP = 10240
NSUB = 16
EPT = EE // NSUB
CHUNK = NP // NSUB
LANES = 16

_MESH = plsc.VectorSubcoreMesh(
    core_axis_name="c", subcore_axis_name="s",
    num_cores=1, num_subcores=NSUB)
_SC_PARAMS = pltpu.CompilerParams(needs_layout_passes=False)


def _zero_vec(ref, n):
    z = jnp.zeros((LANES,), jnp.float32)

    @plsc.parallel_loop(0, n // LANES, unroll=8)
    def _(i):
        ref[pl.ds(i * LANES, LANES)] = z


def _rsqrt16(d):
    i = plsc.bitcast(d, jnp.int32)
    i = jnp.int32(0x5F3759DF) - lax.shift_right_logical(i, 1)
    r = plsc.bitcast(i, jnp.float32)
    for _ in range(3):
        r = r * (1.5 - 0.5 * d * r * r)
    return r


@functools.partial(
    pl.kernel,
    out_type=[jax.ShapeDtypeStruct((NP,), jnp.float32),
              jax.ShapeDtypeStruct((2, NSUB, LANES), jnp.float32),
              jax.ShapeDtypeStruct((NSUB, NP), jnp.float32),
              jax.ShapeDtypeStruct((NSUB, NP), jnp.float32),
              jax.ShapeDtypeStruct((NP,), jnp.float32)],
    mesh=_MESH,
    scratch_types=[pltpu.VMEM((EPT,), jnp.int32),
                   pltpu.VMEM((EPT,), jnp.int32),
                   pltpu.VMEM((NP,), jnp.float32),
                   pltpu.VMEM((NP,), jnp.float32),
                   pltpu.VMEM((NP,), jnp.float32),
                   pltpu.VMEM((NSUB, CHUNK), jnp.float32),
                   pltpu.VMEM((CHUNK,), jnp.float32),
                   pltpu.VMEM((CHUNK,), jnp.float32),
                   pltpu.VMEM((CHUNK,), jnp.float32),
                   pltpu.VMEM((2, LANES), jnp.float32)],
    compiler_params=_SC_PARAMS,
)
def _sc_kernel(src_hbm, dst_hbm, u0_hbm, s_hbm, pa_hbm, pb_hbm, t_hbm,
               src_v, dst_v, t_v, acca_v, accb_v, pst_v, g_v, ns_v, ck_v,
               s_v):
    wid = lax.axis_index("s")
    ebase = wid * EPT
    cbase = wid * CHUNK
    pltpu.sync_copy(src_hbm.at[pl.ds(ebase, EPT)], src_v)
    pltpu.sync_copy(dst_hbm.at[pl.ds(ebase, EPT)], dst_v)

    _zero_vec(acca_v, NP)
    _zero_vec(accb_v, NP)
    ones = jnp.ones((LANES,), jnp.float32)

    @plsc.parallel_loop(0, EPT // LANES, unroll=4)
    def _(i):
        sv = src_v[pl.ds(i * LANES, LANES)]
        dv = dst_v[pl.ds(i * LANES, LANES)]
        plsc.addupdate_scatter(acca_v, [sv], ones)
        plsc.addupdate_scatter(accb_v, [dv], ones)

    pltpu.sync_copy(acca_v, pa_hbm.at[wid])
    pltpu.sync_copy(accb_v, pb_hbm.at[wid])
    plsc.subcore_barrier()

    pltpu.sync_copy(pa_hbm.at[:, pl.ds(cbase, CHUNK)], pst_v)

    def _reduce16(j):
        acc = pst_v[0, pl.ds(j * LANES, LANES)]
        for w in range(1, NSUB):
            acc = acc + pst_v[w, pl.ds(j * LANES, LANES)]
        return acc

    @plsc.parallel_loop(0, CHUNK // LANES, unroll=2)
    def _(j):
        od = _reduce16(j)
        ns_v[pl.ds(j * LANES, LANES)] = _rsqrt16(jnp.maximum(od, 1.0))

    pltpu.sync_copy(pb_hbm.at[:, pl.ds(cbase, CHUNK)], pst_v)

    @plsc.parallel_loop(0, CHUNK // LANES, unroll=2)
    def _(j):
        idg = _reduce16(j)
        nd = _rsqrt16(jnp.maximum(idg, 1.0))
        sl = pl.ds(j * LANES, LANES)
        g_v[sl] = ns_v[sl] * nd
        ck_v[sl] = nd * (1.0 / NN)

    pltpu.sync_copy(ck_v, t_hbm.at[pl.ds(cbase, CHUNK)])
    plsc.subcore_barrier()

    for r in (1, 2, 3):
        pltpu.sync_copy(t_hbm, t_v)
        _zero_vec(acca_v, NP)

        @plsc.parallel_loop(0, EPT // LANES, unroll=4)
        def _(i):
            dv = dst_v[pl.ds(i * LANES, LANES)]
            vals = plsc.load_gather(t_v, [dv])
            sv = src_v[pl.ds(i * LANES, LANES)]
            plsc.addupdate_scatter(acca_v, [sv], vals)

        pltpu.sync_copy(acca_v, pa_hbm.at[wid])
        plsc.subcore_barrier()

        pltpu.sync_copy(pa_hbm.at[:, pl.ds(cbase, CHUNK)], pst_v)
        if r < 3:
            sacc = jnp.zeros((LANES,), jnp.float32)

            @plsc.parallel_loop(0, CHUNK // LANES, unroll=2, carry=sacc)
            def sacc(j, acc):
                p = _reduce16(j)
                sl = pl.ds(j * LANES, LANES)
                ck_v[sl] = g_v[sl] * p
                return acc + ns_v[sl] * p

            s_v[2 - r, :] = sacc
            pltpu.sync_copy(ck_v, t_hbm.at[pl.ds(cbase, CHUNK)])
            plsc.subcore_barrier()
        else:
            @plsc.parallel_loop(0, CHUNK // LANES, unroll=2)
            def _(j):
                p = _reduce16(j)
                sl = pl.ds(j * LANES, LANES)
                ck_v[sl] = ns_v[sl] * p

            pltpu.sync_copy(ck_v, u0_hbm.at[pl.ds(cbase, CHUNK)])

    pltpu.sync_copy(s_v, s_hbm.at[:, wid])


def _head_body(u0_ref, xp_ref, sp_ref, w1_ref, b1_ref, w2_ref,
               b2_ref, w3_ref, b3_ref, wo_ref, bo_ref, out_ref):
    s1 = jnp.sum(sp_ref[1, :, :])
    s2 = jnp.sum(sp_ref[0, :, :])
    y0 = jnp.dot(u0_ref[...], xp_ref[...],
                 preferred_element_type=jnp.float32)
    y1 = jnp.dot(y0, w1_ref[...],
                 preferred_element_type=jnp.float32) + s1 * b1_ref[...]
    y2 = jnp.dot(y1, w2_ref[...],
                 preferred_element_type=jnp.float32) + s2 * b2_ref[...]
    y3 = jnp.dot(y2, w3_ref[...],
                 preferred_element_type=jnp.float32) + b3_ref[...]
    out_ref[...] = jnp.dot(y3, wo_ref[...],
                           preferred_element_type=jnp.float32) + bo_ref[...]


_head_call = pl.pallas_call(
    _head_body,
    out_shape=jax.ShapeDtypeStruct((1, CC), jnp.float32),
)


def kernel(features, edge_index, W1, b1, W2, b2, W3, b3, Wo, bo):
    src = edge_index[0]
    dst = edge_index[1]

    u0, s_parts, _, _, _ = _sc_kernel(src, dst)

    xp = jnp.pad(features, ((0, NP - NN), (0, 0)))
    out = _head_call(u0.reshape(1, NP), xp, s_parts,
                     W1, b1.reshape(1, HH), W2, b2.reshape(1, HH),
                     W3, b3.reshape(1, HH), Wo, bo.reshape(1, CC))
    return out


# speedup vs baseline: 93.5057x; 1.2489x over previous
import functools

import jax
import jax.numpy as jnp
from jax import lax
from jax.experimental import pallas as pl
from jax.experimental.pallas import tpu as pltpu
from jax.experimental.pallas import tpu_sc as plsc

NN = 10000
EE = 320000
DD = 128
HH = 128
CC = 64
NP = 10240
NSUB = 16
EPT = EE // NSUB
EWIN = 20096
CHUNK = NP // NSUB
LANES = 16

_MESH = plsc.VectorSubcoreMesh(
    core_axis_name="c", subcore_axis_name="s",
    num_cores=1, num_subcores=NSUB)
_SC_PARAMS = pltpu.CompilerParams(needs_layout_passes=False)


def _zero_vec(ref, n):
    z = jnp.zeros((LANES,), jnp.float32)

    @plsc.parallel_loop(0, n // LANES, unroll=8)
    def _(i):
        ref[pl.ds(i * LANES, LANES)] = z


def _rsqrt16(d):
    i = plsc.bitcast(d, jnp.int32)
    i = jnp.int32(0x5F3759DF) - lax.shift_right_logical(i, 1)
    r = plsc.bitcast(i, jnp.float32)
    for _ in range(3):
        r = r * (1.5 - 0.5 * d * r * r)
    return r


@functools.partial(
    pl.kernel,
    out_type=[jax.ShapeDtypeStruct((NP,), jnp.float32),
              jax.ShapeDtypeStruct((2, NSUB, LANES), jnp.float32),
              jax.ShapeDtypeStruct((NSUB, NP), jnp.float32),
              jax.ShapeDtypeStruct((NSUB, NP), jnp.float32),
              jax.ShapeDtypeStruct((NP,), jnp.float32)],
    mesh=_MESH,
    scratch_types=[pltpu.VMEM((2, EWIN), jnp.int32),
                   pltpu.VMEM((NP,), jnp.float32),
                   pltpu.VMEM((NP,), jnp.float32),
                   pltpu.VMEM((NP,), jnp.float32),
                   pltpu.VMEM((NSUB, CHUNK), jnp.float32),
                   pltpu.VMEM((NSUB, CHUNK), jnp.float32),
                   pltpu.VMEM((CHUNK,), jnp.float32),
                   pltpu.VMEM((CHUNK,), jnp.float32),
                   pltpu.VMEM((CHUNK,), jnp.float32),
                   pltpu.VMEM((CHUNK,), jnp.float32),
                   pltpu.VMEM((2, LANES), jnp.float32),
                   pltpu.SemaphoreType.DMA,
                   pltpu.SemaphoreType.DMA],
    compiler_params=_SC_PARAMS,
)
def _sc_kernel(ei_hbm, u0_hbm, s_hbm, pa_hbm, pb_hbm, t_hbm,
               ei_v, t_v, acca_v, accb_v, psta_v, pstb_v,
               nd_v, g_v, ns_v, ck_v, s_v, sem1, sem2):
    wid = lax.axis_index("s")
    ebase = wid * EPT
    walign = pl.multiple_of(
        lax.shift_left(lax.shift_right_logical(ebase, 7), 7), 128)
    eoff = ebase - walign
    cbase = wid * CHUNK
    cpe = pltpu.async_copy(ei_hbm.at[:, pl.ds(walign, EWIN)], ei_v, sem1)

    _zero_vec(accb_v, NP)
    cpe.wait()
    ones = jnp.ones((LANES,), jnp.float32)

    @plsc.parallel_loop(0, EPT // LANES, unroll=8)
    def _(i):
        dv = ei_v[1, pl.ds(eoff + i * LANES, LANES)]
        plsc.addupdate_scatter(accb_v, [dv], ones)

    pltpu.sync_copy(accb_v, pb_hbm.at[wid])
    plsc.subcore_barrier()

    cpb = pltpu.async_copy(pb_hbm.at[:, pl.ds(cbase, CHUNK)], pstb_v, sem2)
    _zero_vec(acca_v, NP)
    cpb.wait()

    def _reduce16(pst_v, j):
        acc = pst_v[0, pl.ds(j * LANES, LANES)]
        for w in range(1, NSUB):
            acc = acc + pst_v[w, pl.ds(j * LANES, LANES)]
        return acc

    @plsc.parallel_loop(0, CHUNK // LANES, unroll=2)
    def _(j):
        idg = _reduce16(pstb_v, j)
        nd = _rsqrt16(jnp.maximum(idg, 1.0))
        sl = pl.ds(j * LANES, LANES)
        nd_v[sl] = nd
        ck_v[sl] = nd * (1.0 / NN)

    pltpu.sync_copy(ck_v, t_hbm.at[pl.ds(cbase, CHUNK)])
    plsc.subcore_barrier()

    for r in (1, 2, 3):
        cpt = pltpu.async_copy(t_hbm, t_v, sem1)
        if r > 1:
            _zero_vec(acca_v, NP)
        else:
            _zero_vec(accb_v, NP)
        cpt.wait()

        if r == 1:
            @plsc.parallel_loop(0, EPT // LANES, unroll=8)
            def _(i):
                sl = pl.ds(eoff + i * LANES, LANES)
                dv = ei_v[1, sl]
                vals = plsc.load_gather(t_v, [dv])
                sv = ei_v[0, sl]
                plsc.addupdate_scatter(acca_v, [sv], vals)
                plsc.addupdate_scatter(accb_v, [sv], ones)

            cpa = pltpu.async_copy(acca_v, pa_hbm.at[wid], sem1)
            cpb = pltpu.async_copy(accb_v, pb_hbm.at[wid], sem2)
            cpa.wait()
            cpb.wait()
        else:
            @plsc.parallel_loop(0, EPT // LANES, unroll=8)
            def _(i):
                sl = pl.ds(eoff + i * LANES, LANES)
                dv = ei_v[1, sl]
                vals = plsc.load_gather(t_v, [dv])
                sv = ei_v[0, sl]
                plsc.addupdate_scatter(acca_v, [sv], vals)

            pltpu.sync_copy(acca_v, pa_hbm.at[wid])
        plsc.subcore_barrier()

        cpa = pltpu.async_copy(pa_hbm.at[:, pl.ds(cbase, CHUNK)], psta_v,
                               sem1)
        if r == 1:
            cpb = pltpu.async_copy(pb_hbm.at[:, pl.ds(cbase, CHUNK)], pstb_v,
                                   sem2)
            cpb.wait()
            cpa.wait()

            @plsc.parallel_loop(0, CHUNK // LANES, unroll=2)
            def _(j):
                od = _reduce16(pstb_v, j)
                sl = pl.ds(j * LANES, LANES)
                ns_v[sl] = _rsqrt16(jnp.maximum(od, 1.0))
                g_v[sl] = ns_v[sl] * nd_v[sl]
        else:
            cpa.wait()

        if r < 3:
            sacc = jnp.zeros((LANES,), jnp.float32)

            @plsc.parallel_loop(0, CHUNK // LANES, unroll=2, carry=sacc)
            def sacc(j, acc):
                p = _reduce16(psta_v, j)
                sl = pl.ds(j * LANES, LANES)
                ck_v[sl] = g_v[sl] * p
                return acc + ns_v[sl] * p

            s_v[2 - r, :] = sacc
            pltpu.sync_copy(ck_v, t_hbm.at[pl.ds(cbase, CHUNK)])
            plsc.subcore_barrier()
        else:
            @plsc.parallel_loop(0, CHUNK // LANES, unroll=2)
            def _(j):
                p = _reduce16(psta_v, j)
                sl = pl.ds(j * LANES, LANES)
                ck_v[sl] = ns_v[sl] * p

            pltpu.sync_copy(ck_v, u0_hbm.at[pl.ds(cbase, CHUNK)])

    pltpu.sync_copy(s_v, s_hbm.at[:, wid])


def _head_body(u0_ref, xp_ref, sp_ref, w1_ref, b1_ref, w2_ref,
               b2_ref, w3_ref, b3_ref, wo_ref, bo_ref, out_ref):
    s1 = jnp.sum(sp_ref[1, :, :])
    s2 = jnp.sum(sp_ref[0, :, :])
    y0 = jnp.dot(u0_ref[...], xp_ref[...],
                 preferred_element_type=jnp.float32)
    y1 = jnp.dot(y0, w1_ref[...],
                 preferred_element_type=jnp.float32) + s1 * b1_ref[...]
    y2 = jnp.dot(y1, w2_ref[...],
                 preferred_element_type=jnp.float32) + s2 * b2_ref[...]
    y3 = jnp.dot(y2, w3_ref[...],
                 preferred_element_type=jnp.float32) + b3_ref[...]
    out_ref[...] = jnp.dot(y3, wo_ref[...],
                           preferred_element_type=jnp.float32) + bo_ref[...]


_head_call = pl.pallas_call(
    _head_body,
    out_shape=jax.ShapeDtypeStruct((1, CC), jnp.float32),
)


def kernel(features, edge_index, W1, b1, W2, b2, W3, b3, Wo, bo):
    u0, s_parts, _, _, _ = _sc_kernel(edge_index)

    out = _head_call(u0.reshape(1, NP)[:, :NN], features, s_parts,
                     W1, b1.reshape(1, HH), W2, b2.reshape(1, HH),
                     W3, b3.reshape(1, HH), Wo, bo.reshape(1, CC))
    return out


# speedup vs baseline: 93.9436x; 1.0047x over previous
import functools

import jax
import jax.numpy as jnp
from jax import lax
from jax.experimental import pallas as pl
from jax.experimental.pallas import tpu as pltpu
from jax.experimental.pallas import tpu_sc as plsc

NN = 10000
EE = 320000
DD = 128
HH = 128
CC = 64
NP = 10240
NSUB = 16
EPT = EE // NSUB
EWIN = 20096
CHUNK = NP // NSUB
LANES = 16

_MESH = plsc.VectorSubcoreMesh(
    core_axis_name="c", subcore_axis_name="s",
    num_cores=1, num_subcores=NSUB)
_SC_PARAMS = pltpu.CompilerParams(needs_layout_passes=False)


def _zero_vec(ref, n):
    z = jnp.zeros((LANES,), jnp.float32)

    @plsc.parallel_loop(0, n // LANES, unroll=8)
    def _(i):
        ref[pl.ds(i * LANES, LANES)] = z


def _rsqrt16(d):
    i = plsc.bitcast(d, jnp.int32)
    i = jnp.int32(0x5F3759DF) - lax.shift_right_logical(i, 1)
    r = plsc.bitcast(i, jnp.float32)
    for _ in range(3):
        r = r * (1.5 - 0.5 * d * r * r)
    return r


@functools.partial(
    pl.kernel,
    out_type=[jax.ShapeDtypeStruct((NN,), jnp.float32),
              jax.ShapeDtypeStruct((2, NSUB, LANES), jnp.float32),
              jax.ShapeDtypeStruct((NSUB, NP), jnp.float32),
              jax.ShapeDtypeStruct((NSUB, NP), jnp.float32),
              jax.ShapeDtypeStruct((NP,), jnp.float32)],
    mesh=_MESH,
    scratch_types=[pltpu.VMEM((2, EWIN), jnp.int32),
                   pltpu.VMEM((NP,), jnp.float32),
                   pltpu.VMEM((NP,), jnp.float32),
                   pltpu.VMEM((NP,), jnp.float32),
                   pltpu.VMEM((NSUB, CHUNK), jnp.float32),
                   pltpu.VMEM((NSUB, CHUNK), jnp.float32),
                   pltpu.VMEM((CHUNK,), jnp.float32),
                   pltpu.VMEM((CHUNK,), jnp.float32),
                   pltpu.VMEM((CHUNK,), jnp.float32),
                   pltpu.VMEM((CHUNK,), jnp.float32),
                   pltpu.VMEM((2, LANES), jnp.float32),
                   pltpu.SemaphoreType.DMA,
                   pltpu.SemaphoreType.DMA],
    compiler_params=_SC_PARAMS,
)
def _sc_kernel(ei_hbm, u0_hbm, s_hbm, pa_hbm, pb_hbm, t_hbm,
               ei_v, t_v, acca_v, accb_v, psta_v, pstb_v,
               nd_v, g_v, ns_v, ck_v, s_v, sem1, sem2):
    wid = lax.axis_index("s")
    ebase = wid * EPT
    walign = pl.multiple_of(
        lax.shift_left(lax.shift_right_logical(ebase, 7), 7), 128)
    eoff = ebase - walign
    cbase = wid * CHUNK
    cpe = pltpu.async_copy(ei_hbm.at[:, pl.ds(walign, EWIN)], ei_v, sem1)

    _zero_vec(accb_v, NP)
    cpe.wait()
    ones = jnp.ones((LANES,), jnp.float32)

    @plsc.parallel_loop(0, EPT // LANES, unroll=8)
    def _(i):
        dv = ei_v[1, pl.ds(eoff + i * LANES, LANES)]
        plsc.addupdate_scatter(accb_v, [dv], ones)

    pltpu.sync_copy(accb_v, pb_hbm.at[wid])
    plsc.subcore_barrier()

    cpb = pltpu.async_copy(pb_hbm.at[:, pl.ds(cbase, CHUNK)], pstb_v, sem2)
    _zero_vec(acca_v, NP)
    cpb.wait()

    def _reduce16(pst_v, j):
        acc = pst_v[0, pl.ds(j * LANES, LANES)]
        for w in range(1, NSUB):
            acc = acc + pst_v[w, pl.ds(j * LANES, LANES)]
        return acc

    @plsc.parallel_loop(0, CHUNK // LANES, unroll=2)
    def _(j):
        idg = _reduce16(pstb_v, j)
        nd = _rsqrt16(jnp.maximum(idg, 1.0))
        sl = pl.ds(j * LANES, LANES)
        nd_v[sl] = nd
        ck_v[sl] = nd * (1.0 / NN)

    pltpu.sync_copy(ck_v, t_hbm.at[pl.ds(cbase, CHUNK)])
    plsc.subcore_barrier()

    QT = NP // 4
    for r in (1, 2, 3):
        q0 = lax.rem(wid, 4) * QT
        q1 = lax.rem(wid + 1, 4) * QT
        q2 = lax.rem(wid + 2, 4) * QT
        q3 = lax.rem(wid + 3, 4) * QT
        cps = [pltpu.async_copy(t_hbm.at[pl.ds(q, QT)], t_v.at[pl.ds(q, QT)],
                                sem1) for q in (q0, q1, q2, q3)]
        if r > 1:
            _zero_vec(acca_v, NP)
        else:
            _zero_vec(accb_v, NP)
        for cp in cps:
            cp.wait()

        if r == 1:
            @plsc.parallel_loop(0, EPT // LANES, unroll=8)
            def _(i):
                sl = pl.ds(eoff + i * LANES, LANES)
                dv = ei_v[1, sl]
                vals = plsc.load_gather(t_v, [dv])
                sv = ei_v[0, sl]
                plsc.addupdate_scatter(acca_v, [sv], vals)
                plsc.addupdate_scatter(accb_v, [sv], ones)

            cpa = pltpu.async_copy(acca_v, pa_hbm.at[wid], sem1)
            cpb = pltpu.async_copy(accb_v, pb_hbm.at[wid], sem2)
            cpa.wait()
            cpb.wait()
        else:
            @plsc.parallel_loop(0, EPT // LANES, unroll=8)
            def _(i):
                sl = pl.ds(eoff + i * LANES, LANES)
                dv = ei_v[1, sl]
                vals = plsc.load_gather(t_v, [dv])
                sv = ei_v[0, sl]
                plsc.addupdate_scatter(acca_v, [sv], vals)

            pltpu.sync_copy(acca_v, pa_hbm.at[wid])
        plsc.subcore_barrier()

        cpa = pltpu.async_copy(pa_hbm.at[:, pl.ds(cbase, CHUNK)], psta_v,
                               sem1)
        if r == 1:
            cpb = pltpu.async_copy(pb_hbm.at[:, pl.ds(cbase, CHUNK)], pstb_v,
                                   sem2)
            cpb.wait()
            cpa.wait()

            @plsc.parallel_loop(0, CHUNK // LANES, unroll=2)
            def _(j):
                od = _reduce16(pstb_v, j)
                sl = pl.ds(j * LANES, LANES)
                ns_v[sl] = _rsqrt16(jnp.maximum(od, 1.0))
                g_v[sl] = ns_v[sl] * nd_v[sl]
        else:
            cpa.wait()

        if r < 3:
            sacc = jnp.zeros((LANES,), jnp.float32)

            @plsc.parallel_loop(0, CHUNK // LANES, unroll=2, carry=sacc)
            def sacc(j, acc):
                p = _reduce16(psta_v, j)
                sl = pl.ds(j * LANES, LANES)
                ck_v[sl] = g_v[sl] * p
                return acc + ns_v[sl] * p

            s_v[2 - r, :] = sacc
            pltpu.sync_copy(ck_v, t_hbm.at[pl.ds(cbase, CHUNK)])
            plsc.subcore_barrier()
        else:
            @plsc.parallel_loop(0, CHUNK // LANES, unroll=2)
            def _(j):
                p = _reduce16(psta_v, j)
                sl = pl.ds(j * LANES, LANES)
                ck_v[sl] = ns_v[sl] * p

            @pl.when(wid < NSUB - 1)
            def _():
                pltpu.sync_copy(ck_v, u0_hbm.at[pl.ds(cbase, CHUNK)])

            @pl.when(wid == NSUB - 1)
            def _():
                pltpu.sync_copy(ck_v.at[pl.ds(0, NN - (NSUB - 1) * CHUNK)],
                                u0_hbm.at[pl.ds(cbase, NN - (NSUB - 1) * CHUNK)])

    pltpu.sync_copy(s_v, s_hbm.at[:, wid])


def _head_body(u0_ref, xp_ref, sp_ref, w1_ref, b1_ref, w2_ref,
               b2_ref, w3_ref, b3_ref, wo_ref, bo_ref, out_ref):
    s1 = jnp.sum(sp_ref[1, :, :])
    s2 = jnp.sum(sp_ref[0, :, :])
    y0 = jnp.dot(u0_ref[...], xp_ref[...],
                 preferred_element_type=jnp.float32)
    y1 = jnp.dot(y0, w1_ref[...],
                 preferred_element_type=jnp.float32) + s1 * b1_ref[...]
    y2 = jnp.dot(y1, w2_ref[...],
                 preferred_element_type=jnp.float32) + s2 * b2_ref[...]
    y3 = jnp.dot(y2, w3_ref[...],
                 preferred_element_type=jnp.float32) + b3_ref[...]
    out_ref[...] = jnp.dot(y3, wo_ref[...],
                           preferred_element_type=jnp.float32) + bo_ref[...]


_head_call = pl.pallas_call(
    _head_body,
    out_shape=jax.ShapeDtypeStruct((1, CC), jnp.float32),
)


def kernel(features, edge_index, W1, b1, W2, b2, W3, b3, Wo, bo):
    u0, s_parts, _, _, _ = _sc_kernel(edge_index)

    out = _head_call(u0.reshape(1, NN), features, s_parts,
                     W1, b1.reshape(1, HH), W2, b2.reshape(1, HH),
                     W3, b3.reshape(1, HH), Wo, bo.reshape(1, CC))
    return out


# speedup vs baseline: 98.5720x; 1.0493x over previous
import functools

import jax
import jax.numpy as jnp
from jax import lax
from jax.experimental import pallas as pl
from jax.experimental.pallas import tpu as pltpu
from jax.experimental.pallas import tpu_sc as plsc

NN = 10000
EE = 320000
DD = 128
HH = 128
CC = 64
NP = 10240
NSUB = 16
EPT = EE // NSUB
EWIN = 20096
CHUNK = NP // NSUB
LANES = 16

_MESH = plsc.VectorSubcoreMesh(
    core_axis_name="c", subcore_axis_name="s",
    num_cores=1, num_subcores=NSUB)
_SC_PARAMS = pltpu.CompilerParams(needs_layout_passes=False)


def _zero_vec(ref, n):
    z = jnp.zeros((LANES,), jnp.float32)

    @plsc.parallel_loop(0, n // LANES, unroll=8)
    def _(i):
        ref[pl.ds(i * LANES, LANES)] = z


def _rsqrt16(d):
    i = plsc.bitcast(d, jnp.int32)
    i = jnp.int32(0x5F3759DF) - lax.shift_right_logical(i, 1)
    r = plsc.bitcast(i, jnp.float32)
    for _ in range(3):
        r = r * (1.5 - 0.5 * d * r * r)
    return r


@functools.partial(
    pl.kernel,
    out_type=[jax.ShapeDtypeStruct((NN,), jnp.float32),
              jax.ShapeDtypeStruct((2, NSUB, LANES), jnp.float32),
              jax.ShapeDtypeStruct((NP,), jnp.float32)],
    mesh=_MESH,
    scratch_types=[pltpu.VMEM((2, EWIN), jnp.int32),
                   pltpu.VMEM((NP,), jnp.float32),
                   pltpu.VMEM((NP,), jnp.float32),
                   pltpu.VMEM((NP,), jnp.float32),
                   pltpu.VMEM((NSUB, CHUNK), jnp.float32),
                   pltpu.VMEM((NSUB, CHUNK), jnp.float32),
                   pltpu.VMEM((CHUNK,), jnp.float32),
                   pltpu.VMEM((CHUNK,), jnp.float32),
                   pltpu.VMEM((CHUNK,), jnp.float32),
                   pltpu.VMEM((CHUNK,), jnp.float32),
                   pltpu.VMEM((2, LANES), jnp.float32),
                   pltpu.VMEM_SHARED((NSUB, NP), jnp.float32),
                   pltpu.VMEM_SHARED((NSUB, NP), jnp.float32),
                   pltpu.SemaphoreType.DMA,
                   pltpu.SemaphoreType.DMA],
    compiler_params=_SC_PARAMS,
)
def _sc_kernel(ei_hbm, u0_hbm, s_hbm, t_hbm,
               ei_v, t_v, acca_v, accb_v, psta_v, pstb_v,
               nd_v, g_v, ns_v, ck_v, s_v, pa_hbm, pb_hbm, sem1, sem2):
    wid = lax.axis_index("s")
    ebase = wid * EPT
    walign = pl.multiple_of(
        lax.shift_left(lax.shift_right_logical(ebase, 7), 7), 128)
    eoff = ebase - walign
    cbase = wid * CHUNK
    cpe = pltpu.async_copy(ei_hbm.at[:, pl.ds(walign, EWIN)], ei_v, sem1)

    _zero_vec(accb_v, NP)
    cpe.wait()
    ones = jnp.ones((LANES,), jnp.float32)

    @plsc.parallel_loop(0, EPT // LANES, unroll=8)
    def _(i):
        dv = ei_v[1, pl.ds(eoff + i * LANES, LANES)]
        plsc.addupdate_scatter(accb_v, [dv], ones)

    pltpu.sync_copy(accb_v, pb_hbm.at[wid])
    plsc.subcore_barrier()

    cpb = pltpu.async_copy(pb_hbm.at[:, pl.ds(cbase, CHUNK)], pstb_v, sem2)
    _zero_vec(acca_v, NP)
    cpb.wait()

    def _reduce16(pst_v, j):
        acc = pst_v[0, pl.ds(j * LANES, LANES)]
        for w in range(1, NSUB):
            acc = acc + pst_v[w, pl.ds(j * LANES, LANES)]
        return acc

    @plsc.parallel_loop(0, CHUNK // LANES, unroll=2)
    def _(j):
        idg = _reduce16(pstb_v, j)
        nd = _rsqrt16(jnp.maximum(idg, 1.0))
        sl = pl.ds(j * LANES, LANES)
        nd_v[sl] = nd
        ck_v[sl] = nd * (1.0 / NN)

    pltpu.sync_copy(ck_v, t_hbm.at[pl.ds(cbase, CHUNK)])
    plsc.subcore_barrier()

    QT = NP // 4
    for r in (1, 2, 3):
        q0 = lax.rem(wid, 4) * QT
        q1 = lax.rem(wid + 1, 4) * QT
        q2 = lax.rem(wid + 2, 4) * QT
        q3 = lax.rem(wid + 3, 4) * QT
        cps = [pltpu.async_copy(t_hbm.at[pl.ds(q, QT)], t_v.at[pl.ds(q, QT)],
                                sem1) for q in (q0, q1, q2, q3)]
        if r > 1:
            _zero_vec(acca_v, NP)
        else:
            _zero_vec(accb_v, NP)
        for cp in cps:
            cp.wait()

        if r == 1:
            @plsc.parallel_loop(0, EPT // LANES, unroll=8)
            def _(i):
                sl = pl.ds(eoff + i * LANES, LANES)
                dv = ei_v[1, sl]
                vals = plsc.load_gather(t_v, [dv])
                sv = ei_v[0, sl]
                plsc.addupdate_scatter(acca_v, [sv], vals)
                plsc.addupdate_scatter(accb_v, [sv], ones)

            cpa = pltpu.async_copy(acca_v, pa_hbm.at[wid], sem1)
            cpb = pltpu.async_copy(accb_v, pb_hbm.at[wid], sem2)
            cpa.wait()
            cpb.wait()
        else:
            @plsc.parallel_loop(0, EPT // LANES, unroll=8)
            def _(i):
                sl = pl.ds(eoff + i * LANES, LANES)
                dv = ei_v[1, sl]
                vals = plsc.load_gather(t_v, [dv])
                sv = ei_v[0, sl]
                plsc.addupdate_scatter(acca_v, [sv], vals)

            pltpu.sync_copy(acca_v, pa_hbm.at[wid])
        plsc.subcore_barrier()

        cpa = pltpu.async_copy(pa_hbm.at[:, pl.ds(cbase, CHUNK)], psta_v,
                               sem1)
        if r == 1:
            cpb = pltpu.async_copy(pb_hbm.at[:, pl.ds(cbase, CHUNK)], pstb_v,
                                   sem2)
            cpb.wait()
            cpa.wait()

            @plsc.parallel_loop(0, CHUNK // LANES, unroll=2)
            def _(j):
                od = _reduce16(pstb_v, j)
                sl = pl.ds(j * LANES, LANES)
                ns_v[sl] = _rsqrt16(jnp.maximum(od, 1.0))
                g_v[sl] = ns_v[sl] * nd_v[sl]
        else:
            cpa.wait()

        if r < 3:
            sacc = jnp.zeros((LANES,), jnp.float32)

            @plsc.parallel_loop(0, CHUNK // LANES, unroll=2, carry=sacc)
            def sacc(j, acc):
                p = _reduce16(psta_v, j)
                sl = pl.ds(j * LANES, LANES)
                ck_v[sl] = g_v[sl] * p
                return acc + ns_v[sl] * p

            s_v[2 - r, :] = sacc
            pltpu.sync_copy(ck_v, t_hbm.at[pl.ds(cbase, CHUNK)])
            plsc.subcore_barrier()
        else:
            @plsc.parallel_loop(0, CHUNK // LANES, unroll=2)
            def _(j):
                p = _reduce16(psta_v, j)
                sl = pl.ds(j * LANES, LANES)
                ck_v[sl] = ns_v[sl] * p

            @pl.when(wid < NSUB - 1)
            def _():
                pltpu.sync_copy(ck_v, u0_hbm.at[pl.ds(cbase, CHUNK)])

            @pl.when(wid == NSUB - 1)
            def _():
                pltpu.sync_copy(ck_v.at[pl.ds(0, NN - (NSUB - 1) * CHUNK)],
                                u0_hbm.at[pl.ds(cbase, NN - (NSUB - 1) * CHUNK)])

    pltpu.sync_copy(s_v, s_hbm.at[:, wid])


def _head_body(u0_ref, xp_ref, sp_ref, w1_ref, b1_ref, w2_ref,
               b2_ref, w3_ref, b3_ref, wo_ref, bo_ref, out_ref):
    s1 = jnp.sum(sp_ref[1, :, :])
    s2 = jnp.sum(sp_ref[0, :, :])
    y0 = jnp.dot(u0_ref[...], xp_ref[...],
                 preferred_element_type=jnp.float32)
    y1 = jnp.dot(y0, w1_ref[...],
                 preferred_element_type=jnp.float32) + s1 * b1_ref[...]
    y2 = jnp.dot(y1, w2_ref[...],
                 preferred_element_type=jnp.float32) + s2 * b2_ref[...]
    y3 = jnp.dot(y2, w3_ref[...],
                 preferred_element_type=jnp.float32) + b3_ref[...]
    out_ref[...] = jnp.dot(y3, wo_ref[...],
                           preferred_element_type=jnp.float32) + bo_ref[...]


_head_call = pl.pallas_call(
    _head_body,
    out_shape=jax.ShapeDtypeStruct((1, CC), jnp.float32),
)


def kernel(features, edge_index, W1, b1, W2, b2, W3, b3, Wo, bo):
    u0, s_parts, _ = _sc_kernel(edge_index)

    out = _head_call(u0.reshape(1, NN), features, s_parts,
                     W1, b1.reshape(1, HH), W2, b2.reshape(1, HH),
                     W3, b3.reshape(1, HH), Wo, bo.reshape(1, CC))
    return out
